# Initial kernel scaffold; baseline (speedup 1.0000x reference)
import jax
import jax.numpy as jnp
from jax.experimental import pallas as pl


def kernel(E_u_0, E_i_0, vals, ut, vt, u_mul_s, v_mul_s, pos_api_emb, neg_api_emb, mashup_emb, W_api, b_api, W_mashup, b_mashup, rows, cols, uids, pos, neg):
    raise NotImplementedError("write your pallas kernel here")


# speedup vs baseline: 2.5777x; 2.5777x over previous
import functools

import jax
import jax.numpy as jnp
from jax import lax
from jax.experimental import pallas as pl
from jax.experimental.pallas import tpu as pltpu
from jax.experimental.pallas import tpu_sc as plsc

_N = 10000
_D = 128
_E = 320000
_TEMP = 0.2
_LAM1 = 0.2
_LAM2 = 1e-07
_B = 4096

_NC = 2
_NS = 16
_LANES = 16
_EPT = _E // _NS
_CHUNK = 80
_NSTEP = _EPT // _CHUNK
_NP = 10240
_RPT = _NP // _NS

_f32 = jnp.float32
_i32 = jnp.int32


def _spmm_body(tbl, gidx, sidx, vals, zeros, out,
               gix_v, six_v, val_v, rows_v, acc_sh, sem):
    c = lax.axis_index("c")
    s = lax.axis_index("s")
    r0 = s * _RPT
    pltpu.sync_copy(zeros.at[pl.ds(r0, _RPT)], acc_sh.at[pl.ds(r0, _RPT)])
    plsc.subcore_barrier()

    goff = jnp.broadcast_to((c * _NP).astype(_i32), (_LANES,))

    @pl.loop(0, _NSTEP)
    def _chunk(i):
        base = c * _E + s * _EPT + i * _CHUNK
        vbase = s * _EPT + i * _CHUNK
        pltpu.sync_copy(gidx.at[pl.ds(base, _CHUNK)], gix_v)
        pltpu.sync_copy(sidx.at[pl.ds(base, _CHUNK)], six_v)
        pltpu.sync_copy(vals.at[pl.ds(vbase, _CHUNK)], val_v)
        for g in range(_CHUNK // _LANES):
            sl = pl.ds(g * _LANES, _LANES)
            gix_v[sl] = gix_v[sl] + goff
        pltpu.async_copy(tbl.at[gix_v], rows_v, sem).wait()
        for e in range(_CHUNK):
            lane = jnp.full((_LANES,), e, _i32)
            vb = plsc.load_gather(val_v, [lane])
            for cc in range(_D // _LANES):
                sl = pl.ds(cc * _LANES, _LANES)
                rows_v[e, sl] = rows_v[e, sl] * vb
        pltpu.sync_copy(rows_v, acc_sh.at[six_v], add=True)

    plsc.subcore_barrier()
    woff = (1 - c) * _NP
    pltpu.sync_copy(acc_sh.at[pl.ds(r0, _RPT)], out.at[pl.ds(woff + r0, _RPT)])


_spmm = functools.partial(
    pl.kernel,
    out_type=jax.ShapeDtypeStruct((2 * _NP, _D), _f32),
    mesh=plsc.VectorSubcoreMesh(core_axis_name="c", subcore_axis_name="s"),
    scratch_types=[
        pltpu.VMEM((_CHUNK,), _i32),
        pltpu.VMEM((_CHUNK,), _i32),
        pltpu.VMEM((_CHUNK,), _f32),
        pltpu.VMEM((_CHUNK, _D), _f32),
        pltpu.VMEM_SHARED((_NP, _D), _f32),
        pltpu.SemaphoreType.DMA,
    ],
    compiler_params=pltpu.CompilerParams(needs_layout_passes=False),
)(_spmm_body)


_GROWS = 6 * _B
_GPW = _GROWS // (_NC * _NS)
_GCHUNK = 128
_GSTEPS = _GPW // _GCHUNK


def _gather_body(tbl, idx, out, idx_v, buf_v, sem):
    c = lax.axis_index("c")
    s = lax.axis_index("s")
    wid = s * _NC + c

    @pl.loop(0, _GSTEPS)
    def _step(k):
        base = wid * _GPW + k * _GCHUNK
        pltpu.sync_copy(idx.at[pl.ds(base, _GCHUNK)], idx_v)
        pltpu.async_copy(tbl.at[idx_v], buf_v, sem).wait()
        pltpu.sync_copy(buf_v, out.at[pl.ds(base, _GCHUNK)])


_gather = functools.partial(
    pl.kernel,
    out_type=jax.ShapeDtypeStruct((_GROWS, _D), _f32),
    mesh=plsc.VectorSubcoreMesh(core_axis_name="c", subcore_axis_name="s"),
    scratch_types=[
        pltpu.VMEM((_GCHUNK,), _i32),
        pltpu.VMEM((_GCHUNK, _D), _f32),
        pltpu.SemaphoreType.DMA,
    ],
)(_gather_body)


def _dense_small_body(eu0, ei0, zu1, zi1, ut, vt, wapi, bapi, wm, bm,
                      wu_out, wi_out, reg_out):
    wu_out[...] = jnp.dot(vt[...], ei0[...] + zi1[...],
                          preferred_element_type=_f32)
    wi_out[...] = jnp.dot(ut[...], eu0[...] + zu1[...],
                          preferred_element_type=_f32)
    reg = (jnp.sum(eu0[...] * eu0[...]) + jnp.sum(ei0[...] * ei0[...])
           + jnp.sum(wapi[...] * wapi[...]) + jnp.sum(bapi[...] * bapi[...])
           + jnp.sum(wm[...] * wm[...]) + jnp.sum(bm[...] * bm[...]))
    reg_out[...] = jnp.reshape(reg * _LAM2, (1, 1))


_dense_small = pl.pallas_call(
    _dense_small_body,
    out_shape=[
        jax.ShapeDtypeStruct((64, _D), _f32),
        jax.ShapeDtypeStruct((64, _D), _f32),
        jax.ShapeDtypeStruct((1, 1), _f32),
    ],
)


_ABLK = 1000


def _assemble_body(eu0, ei0, zu1, zi1, zu2, zi2, umul, vmul, wu, wi,
                   eu_out, ei_out, gu_out, gi_out):
    eu_out[...] = eu0[...] + zu1[...] + zu2[...]
    ei_out[...] = ei0[...] + zi1[...] + zi2[...]
    gu_out[...] = eu0[...] + jnp.dot(umul[...], wu[...],
                                     preferred_element_type=_f32)
    gi_out[...] = ei0[...] + jnp.dot(vmul[...], wi[...],
                                     preferred_element_type=_f32)


def _assemble(eu0, ei0, zu1, zi1, zu2, zi2, umul, vmul, wu, wi):
    blk = lambda w: pl.BlockSpec((_ABLK, w), lambda i: (i, 0))
    full = pl.BlockSpec((64, _D), lambda i: (0, 0))
    return pl.pallas_call(
        _assemble_body,
        grid=(_N // _ABLK,),
        in_specs=[blk(_D)] * 6 + [blk(64), blk(64), full, full],
        out_specs=[blk(_D)] * 4,
        out_shape=[jax.ShapeDtypeStruct((_N, _D), _f32)] * 4,
    )(eu0, ei0, zu1, zi1, zu2, zi2, umul, vmul, wu, wi)


_JB = 400
_NJ = _N // _JB


def _loss_body(eu_blk, ei_blk, gu_b, gi_b, eu_b, eip, ein, reg,
               loss_out, lr_out, ls_out, acc_u, acc_i):
    j = pl.program_id(0)

    @pl.when(j == 0)
    def _init():
        acc_u[...] = jnp.zeros_like(acc_u)
        acc_i[...] = jnp.zeros_like(acc_i)

    dn = (((1,), (1,)), ((), ()))
    mu = lax.dot_general(gu_b[...], eu_blk[...], dn,
                         preferred_element_type=_f32)
    acc_u[...] += jnp.sum(jnp.exp(mu / _TEMP), axis=1, keepdims=True)
    mi = lax.dot_general(gi_b[...], ei_blk[...], dn,
                         preferred_element_type=_f32)
    acc_i[...] += jnp.sum(jnp.exp(mi / _TEMP), axis=1, keepdims=True)

    @pl.when(j == _NJ - 1)
    def _final():
        neg_score = (jnp.mean(jnp.log(acc_u[...] + 1e-08))
                     + jnp.mean(jnp.log(acc_i[...] + 1e-08)))
        gu = gu_b[...]
        eu = eu_b[...]
        ei_cat = jnp.concatenate([eip[...], ein[...]], axis=0)
        pos_score = (jnp.mean(jnp.clip(jnp.sum(gu * eu, 1) / _TEMP, -5.0, 5.0))
                     + jnp.mean(jnp.clip(jnp.sum(gi_b[...] * ei_cat, 1) / _TEMP,
                                         -5.0, 5.0)))
        loss_s = -pos_score + neg_score
        ps = jnp.sum(eu * eip[...], 1)
        ns = jnp.sum(eu * ein[...], 1)
        loss_r = jnp.mean(jnp.log(1.0 + jnp.exp(ns - ps)))
        lam_ls = _LAM1 * loss_s
        ls_out[...] = jnp.reshape(lam_ls, (1, 1))
        lr_out[...] = jnp.reshape(loss_r, (1, 1))
        loss_out[...] = jnp.reshape(loss_r + lam_ls, (1, 1)) + reg[...]


def _loss(E_u, E_i, gu_b, gi_b, eu_b, eip, ein, reg):
    blk = pl.BlockSpec((_JB, _D), lambda j: (j, 0))
    fullb = lambda r: pl.BlockSpec((r, _D), lambda j: (0, 0))
    one = pl.BlockSpec((1, 1), lambda j: (0, 0))
    return pl.pallas_call(
        _loss_body,
        grid=(_NJ,),
        in_specs=[blk, blk, fullb(_B), fullb(2 * _B), fullb(_B), fullb(_B),
                  fullb(_B), one],
        out_specs=[one, one, one],
        out_shape=[jax.ShapeDtypeStruct((1, 1), _f32)] * 3,
        scratch_shapes=[pltpu.VMEM((_B, 1), _f32),
                        pltpu.VMEM((2 * _B, 1), _f32)],
    )(E_u, E_i, gu_b, gi_b, eu_b, eip, ein, reg)


def kernel(E_u_0, E_i_0, vals, ut, vt, u_mul_s, v_mul_s,
           pos_api_emb, neg_api_emb, mashup_emb,
           W_api, b_api, W_mashup, b_mashup,
           rows, cols, uids, pos, neg):
    rows = rows.astype(_i32)
    cols = cols.astype(_i32)
    gidx = jnp.concatenate([cols, rows])
    sidx = jnp.concatenate([rows, cols])
    zeros = jnp.zeros((_NP, _D), _f32)
    pad = jnp.zeros((_NP - _N, _D), _f32)

    T0 = jnp.concatenate([E_i_0, pad, E_u_0, pad], axis=0)
    Z1 = _spmm(T0, gidx, sidx, vals, zeros)
    Z2 = _spmm(Z1, gidx, sidx, vals, zeros)
    zi1, zu1 = Z1[:_N], Z1[_NP:_NP + _N]
    zi2, zu2 = Z2[:_N], Z2[_NP:_NP + _N]

    wu, wi, reg = _dense_small(E_u_0, E_i_0, zu1, zi1, ut, vt,
                               W_api, b_api.reshape(1, _D),
                               W_mashup, b_mashup.reshape(1, _D))
    E_u, E_i, G_u, G_i = _assemble(E_u_0, E_i_0, zu1, zi1, zu2, zi2,
                                   u_mul_s, v_mul_s, wu, wi)

    T4 = jnp.concatenate([G_u, E_u, G_i, E_i], axis=0)
    u32 = uids.astype(_i32)
    p32 = pos.astype(_i32)
    n32 = neg.astype(_i32)
    gidx2 = jnp.concatenate([u32, u32 + _N, p32 + 2 * _N, n32 + 2 * _N,
                             p32 + 3 * _N, n32 + 3 * _N])
    rows_g = _gather(T4, gidx2)
    gu_b = rows_g[:_B]
    eu_b = rows_g[_B:2 * _B]
    gi_b = rows_g[2 * _B:4 * _B]
    eip = rows_g[4 * _B:5 * _B]
    ein = rows_g[5 * _B:]

    loss, loss_r, lam_ls = _loss(E_u, E_i, gu_b, gi_b, eu_b, eip, ein, reg)
    return (loss.reshape(()), loss_r.reshape(()), lam_ls.reshape(()),
            mashup_emb, pos_api_emb, neg_api_emb, E_u, E_i)


# speedup vs baseline: 2.6576x; 1.0310x over previous
import functools

import jax
import jax.numpy as jnp
from jax import lax
from jax.experimental import pallas as pl
from jax.experimental.pallas import tpu as pltpu
from jax.experimental.pallas import tpu_sc as plsc

_N = 10000
_D = 128
_E = 320000
_TEMP = 0.2
_LAM1 = 0.2
_LAM2 = 1e-07
_B = 4096

_NC = 2
_NS = 16
_LANES = 16
_EPT = _E // _NS
_CHUNK = 128
_SUB = 16
_NSUPER = 10
_NSTEP = _SUB * _NSUPER
_EPAD = _NSTEP * _CHUNK
_NP = 10240
_RPT = _NP // _NS

_f32 = jnp.float32
_i32 = jnp.int32


def _spmm_body(tbl, gidx, sidx, vals, zeros, out,
               gix_v, six_v, val_v, buf0, buf1, acc_sh, sem0, sem1):
    c = lax.axis_index("c")
    s = lax.axis_index("s")
    r0 = s * _RPT
    pltpu.sync_copy(zeros.at[pl.ds(r0, _RPT)], acc_sh.at[pl.ds(r0, _RPT)])
    plsc.subcore_barrier()

    bufs = (buf0, buf1)
    sems = (sem0, sem1)

    @pl.loop(0, _NSUPER)
    def _super(k):
        base = k * _SUB
        pltpu.sync_copy(gidx.at[c, s, pl.ds(base, _SUB)], gix_v)
        pltpu.sync_copy(sidx.at[c, s, pl.ds(base, _SUB)], six_v)
        pltpu.sync_copy(vals.at[s, pl.ds(base, _SUB)], val_v)
        pltpu.async_copy(tbl.at[gix_v.at[0]], buf0, sem0)
        pltpu.async_copy(tbl.at[gix_v.at[1]], buf1, sem1)

        @pl.loop(0, _SUB // 2)
        def _inner(i):
            for b in range(2):
                step = i * 2 + b
                buf = bufs[b]
                pltpu.make_async_copy(tbl.at[gix_v.at[step]], buf,
                                      sems[b]).wait()
                row_sp = jnp.broadcast_to(step, (_LANES,)).astype(_i32)
                for e in range(_CHUNK):
                    lane = jnp.full((_LANES,), e, _i32)
                    vb = plsc.load_gather(val_v, [row_sp, lane])
                    for cc in range(_D // _LANES):
                        sl = pl.ds(cc * _LANES, _LANES)
                        buf[e, sl] = buf[e, sl] * vb
                pltpu.sync_copy(buf, acc_sh.at[six_v.at[step]], add=True)
                nxt = step + 2
                nxt = jnp.where(nxt >= _SUB, nxt - _SUB, nxt)
                pltpu.async_copy(tbl.at[gix_v.at[nxt]], buf, sems[b])

        for b in range(2):
            pltpu.make_async_copy(tbl.at[gix_v.at[b]], bufs[b], sems[b]).wait()

    plsc.subcore_barrier()
    woff = (1 - c) * _NP
    pltpu.sync_copy(acc_sh.at[pl.ds(r0, _RPT)], out.at[pl.ds(woff + r0, _RPT)])


_spmm = functools.partial(
    pl.kernel,
    out_type=jax.ShapeDtypeStruct((2 * _NP, _D), _f32),
    mesh=plsc.VectorSubcoreMesh(core_axis_name="c", subcore_axis_name="s"),
    scratch_types=[
        pltpu.VMEM((_SUB, _CHUNK), _i32),
        pltpu.VMEM((_SUB, _CHUNK), _i32),
        pltpu.VMEM((_SUB, _CHUNK), _f32),
        pltpu.VMEM((_CHUNK, _D), _f32),
        pltpu.VMEM((_CHUNK, _D), _f32),
        pltpu.VMEM_SHARED((_NP, _D), _f32),
        pltpu.SemaphoreType.DMA,
        pltpu.SemaphoreType.DMA,
    ],
    compiler_params=pltpu.CompilerParams(needs_layout_passes=False),
)(_spmm_body)


_GROWS = 6 * _B
_GPW = _GROWS // (_NC * _NS)
_GCHUNK = 128
_GSTEPS = _GPW // _GCHUNK


def _gather_body(tbl, idx, out, idx_v, buf_v, sem):
    c = lax.axis_index("c")
    s = lax.axis_index("s")
    wid = s * _NC + c

    @pl.loop(0, _GSTEPS)
    def _step(k):
        base = wid * _GPW + k * _GCHUNK
        pltpu.sync_copy(idx.at[pl.ds(base, _GCHUNK)], idx_v)
        pltpu.async_copy(tbl.at[idx_v], buf_v, sem).wait()
        pltpu.sync_copy(buf_v, out.at[pl.ds(base, _GCHUNK)])


_gather = functools.partial(
    pl.kernel,
    out_type=jax.ShapeDtypeStruct((_GROWS, _D), _f32),
    mesh=plsc.VectorSubcoreMesh(core_axis_name="c", subcore_axis_name="s"),
    scratch_types=[
        pltpu.VMEM((_GCHUNK,), _i32),
        pltpu.VMEM((_GCHUNK, _D), _f32),
        pltpu.SemaphoreType.DMA,
    ],
)(_gather_body)


def _dense_small_body(eu0, ei0, zu1, zi1, ut, vt, wapi, bapi, wm, bm,
                      wu_out, wi_out, reg_out):
    wu_out[...] = jnp.dot(vt[...], ei0[...] + zi1[...],
                          preferred_element_type=_f32)
    wi_out[...] = jnp.dot(ut[...], eu0[...] + zu1[...],
                          preferred_element_type=_f32)
    reg = (jnp.sum(eu0[...] * eu0[...]) + jnp.sum(ei0[...] * ei0[...])
           + jnp.sum(wapi[...] * wapi[...]) + jnp.sum(bapi[...] * bapi[...])
           + jnp.sum(wm[...] * wm[...]) + jnp.sum(bm[...] * bm[...]))
    reg_out[...] = jnp.reshape(reg * _LAM2, (1, 1))


_dense_small = pl.pallas_call(
    _dense_small_body,
    out_shape=[
        jax.ShapeDtypeStruct((64, _D), _f32),
        jax.ShapeDtypeStruct((64, _D), _f32),
        jax.ShapeDtypeStruct((1, 1), _f32),
    ],
)


_ABLK = 1000


def _assemble_body(eu0, ei0, zu1, zi1, zu2, zi2, umul, vmul, wu, wi,
                   eu_out, ei_out, gu_out, gi_out):
    eu_out[...] = eu0[...] + zu1[...] + zu2[...]
    ei_out[...] = ei0[...] + zi1[...] + zi2[...]
    gu_out[...] = eu0[...] + jnp.dot(umul[...], wu[...],
                                     preferred_element_type=_f32)
    gi_out[...] = ei0[...] + jnp.dot(vmul[...], wi[...],
                                     preferred_element_type=_f32)


def _assemble(eu0, ei0, zu1, zi1, zu2, zi2, umul, vmul, wu, wi):
    blk = lambda w: pl.BlockSpec((_ABLK, w), lambda i: (i, 0))
    full = pl.BlockSpec((64, _D), lambda i: (0, 0))
    return pl.pallas_call(
        _assemble_body,
        grid=(_N // _ABLK,),
        in_specs=[blk(_D)] * 6 + [blk(64), blk(64), full, full],
        out_specs=[blk(_D)] * 4,
        out_shape=[jax.ShapeDtypeStruct((_N, _D), _f32)] * 4,
    )(eu0, ei0, zu1, zi1, zu2, zi2, umul, vmul, wu, wi)


_JB = 400
_NJ = _N // _JB


def _loss_body(eu_blk, ei_blk, gu_b, gi_b, eu_b, eip, ein, reg,
               loss_out, lr_out, ls_out, acc_u, acc_i):
    j = pl.program_id(0)

    @pl.when(j == 0)
    def _init():
        acc_u[...] = jnp.zeros_like(acc_u)
        acc_i[...] = jnp.zeros_like(acc_i)

    dn = (((1,), (1,)), ((), ()))
    mu = lax.dot_general(gu_b[...], eu_blk[...], dn,
                         preferred_element_type=_f32)
    acc_u[...] += jnp.sum(jnp.exp(mu / _TEMP), axis=1, keepdims=True)
    mi = lax.dot_general(gi_b[...], ei_blk[...], dn,
                         preferred_element_type=_f32)
    acc_i[...] += jnp.sum(jnp.exp(mi / _TEMP), axis=1, keepdims=True)

    @pl.when(j == _NJ - 1)
    def _final():
        neg_score = (jnp.mean(jnp.log(acc_u[...] + 1e-08))
                     + jnp.mean(jnp.log(acc_i[...] + 1e-08)))
        gu = gu_b[...]
        eu = eu_b[...]
        ei_cat = jnp.concatenate([eip[...], ein[...]], axis=0)
        pos_score = (jnp.mean(jnp.clip(jnp.sum(gu * eu, 1) / _TEMP, -5.0, 5.0))
                     + jnp.mean(jnp.clip(jnp.sum(gi_b[...] * ei_cat, 1) / _TEMP,
                                         -5.0, 5.0)))
        loss_s = -pos_score + neg_score
        ps = jnp.sum(eu * eip[...], 1)
        ns = jnp.sum(eu * ein[...], 1)
        loss_r = jnp.mean(jnp.log(1.0 + jnp.exp(ns - ps)))
        lam_ls = _LAM1 * loss_s
        ls_out[...] = jnp.reshape(lam_ls, (1, 1))
        lr_out[...] = jnp.reshape(loss_r, (1, 1))
        loss_out[...] = jnp.reshape(loss_r + lam_ls, (1, 1)) + reg[...]


def _loss(E_u, E_i, gu_b, gi_b, eu_b, eip, ein, reg):
    blk = pl.BlockSpec((_JB, _D), lambda j: (j, 0))
    fullb = lambda r: pl.BlockSpec((r, _D), lambda j: (0, 0))
    one = pl.BlockSpec((1, 1), lambda j: (0, 0))
    return pl.pallas_call(
        _loss_body,
        grid=(_NJ,),
        in_specs=[blk, blk, fullb(_B), fullb(2 * _B), fullb(_B), fullb(_B),
                  fullb(_B), one],
        out_specs=[one, one, one],
        out_shape=[jax.ShapeDtypeStruct((1, 1), _f32)] * 3,
        scratch_shapes=[pltpu.VMEM((_B, 1), _f32),
                        pltpu.VMEM((2 * _B, 1), _f32)],
    )(E_u, E_i, gu_b, gi_b, eu_b, eip, ein, reg)


def kernel(E_u_0, E_i_0, vals, ut, vt, u_mul_s, v_mul_s,
           pos_api_emb, neg_api_emb, mashup_emb,
           W_api, b_api, W_mashup, b_mashup,
           rows, cols, uids, pos, neg):
    rows = rows.astype(_i32)
    cols = cols.astype(_i32)

    def _slab(x, fill):
        x = x.reshape(_NS, _EPT)
        x = jnp.pad(x, ((0, 0), (0, _EPAD - _EPT)), constant_values=fill)
        return x.reshape(_NS, _NSTEP, _CHUNK)

    gidx = jnp.stack([_slab(cols, 0), _slab(rows, 0) + _NP])
    sidx = jnp.stack([_slab(rows, 0), _slab(cols, 0)])
    vals_t = _slab(vals, 0.0)
    zeros = jnp.zeros((_NP, _D), _f32)
    pad = jnp.zeros((_NP - _N, _D), _f32)

    T0 = jnp.concatenate([E_i_0, pad, E_u_0, pad], axis=0)
    Z1 = _spmm(T0, gidx, sidx, vals_t, zeros)
    Z2 = _spmm(Z1, gidx, sidx, vals_t, zeros)
    zi1, zu1 = Z1[:_N], Z1[_NP:_NP + _N]
    zi2, zu2 = Z2[:_N], Z2[_NP:_NP + _N]

    wu, wi, reg = _dense_small(E_u_0, E_i_0, zu1, zi1, ut, vt,
                               W_api, b_api.reshape(1, _D),
                               W_mashup, b_mashup.reshape(1, _D))
    E_u, E_i, G_u, G_i = _assemble(E_u_0, E_i_0, zu1, zi1, zu2, zi2,
                                   u_mul_s, v_mul_s, wu, wi)

    T4 = jnp.concatenate([G_u, E_u, G_i, E_i], axis=0)
    u32 = uids.astype(_i32)
    p32 = pos.astype(_i32)
    n32 = neg.astype(_i32)
    gidx2 = jnp.concatenate([u32, u32 + _N, p32 + 2 * _N, n32 + 2 * _N,
                             p32 + 3 * _N, n32 + 3 * _N])
    rows_g = _gather(T4, gidx2)
    gu_b = rows_g[:_B]
    eu_b = rows_g[_B:2 * _B]
    gi_b = rows_g[2 * _B:4 * _B]
    eip = rows_g[4 * _B:5 * _B]
    ein = rows_g[5 * _B:]

    loss, loss_r, lam_ls = _loss(E_u, E_i, gu_b, gi_b, eu_b, eip, ein, reg)
    return (loss.reshape(()), loss_r.reshape(()), lam_ls.reshape(()),
            mashup_emb, pos_api_emb, neg_api_emb, E_u, E_i)


# speedup vs baseline: 3.0348x; 1.1419x over previous
import functools

import jax
import jax.numpy as jnp
from jax import lax
from jax.experimental import pallas as pl
from jax.experimental.pallas import tpu as pltpu
from jax.experimental.pallas import tpu_sc as plsc

_N = 10000
_D = 128
_E = 320000
_TEMP = 0.2
_LAM1 = 0.2
_LAM2 = 1e-07
_B = 4096

_NC = 2
_NS = 16
_LANES = 16
_EPT = _E // _NS
_CHUNK = 64
_SUB = 32
_NSUPER = 10
_NSTEP = _SUB * _NSUPER
_EPAD = _NSTEP * _CHUNK
_NP = 10240
_RPT = _NP // _NS

_f32 = jnp.float32
_i32 = jnp.int32


def _spmm_body(tbl, gidx, sidx, vals, zeros, out,
               gix_v, six_v, val_v, g0, g1, s0, s1, acc_sh,
               gsem0, gsem1, ssem0, ssem1):
    c = lax.axis_index("c")
    s = lax.axis_index("s")
    r0 = s * _RPT
    pltpu.sync_copy(zeros.at[pl.ds(r0, _RPT)], acc_sh.at[pl.ds(r0, _RPT)])
    plsc.subcore_barrier()

    gbufs = (g0, g1)
    sbufs = (s0, s1)
    gsems = (gsem0, gsem1)
    ssems = (ssem0, ssem1)

    def _scale(gbuf, sbuf, step):
        row_sp = jnp.broadcast_to(step, (_LANES,)).astype(_i32)
        for e in range(_CHUNK):
            lane = jnp.full((_LANES,), e, _i32)
            vb = plsc.load_gather(val_v, [row_sp, lane])
            for cc in range(_D // _LANES):
                sl = pl.ds(cc * _LANES, _LANES)
                sbuf[e, sl] = gbuf[e, sl] * vb

    @pl.loop(0, _NSUPER)
    def _super(k):
        base = k * _SUB
        pltpu.sync_copy(gidx.at[c, s, pl.ds(base, _SUB)], gix_v)
        pltpu.sync_copy(sidx.at[c, s, pl.ds(base, _SUB)], six_v)
        pltpu.sync_copy(vals.at[s, pl.ds(base, _SUB)], val_v)
        pltpu.async_copy(tbl.at[gix_v.at[0]], g0, gsem0)
        pltpu.async_copy(tbl.at[gix_v.at[1]], g1, gsem1)

        for b in range(2):
            pltpu.make_async_copy(tbl.at[gix_v.at[b]], gbufs[b],
                                  gsems[b]).wait()
            _scale(gbufs[b], sbufs[b], b)
            pltpu.async_copy(tbl.at[gix_v.at[b + 2]], gbufs[b], gsems[b])
            pltpu.async_copy(sbufs[b], acc_sh.at[six_v.at[b]], ssems[b],
                             add=True)

        @pl.loop(1, _SUB // 2)
        def _inner(i):
            for b in range(2):
                step = i * 2 + b
                gbuf, sbuf = gbufs[b], sbufs[b]
                pltpu.make_async_copy(tbl.at[gix_v.at[step]], gbuf,
                                      gsems[b]).wait()
                pltpu.make_async_copy(sbuf, acc_sh.at[six_v.at[step]],
                                      ssems[b]).wait()
                _scale(gbuf, sbuf, step)
                nxt = step + 2
                nxt = jnp.where(nxt >= _SUB, nxt - _SUB, nxt)
                pltpu.async_copy(tbl.at[gix_v.at[nxt]], gbuf, gsems[b])
                pltpu.async_copy(sbuf, acc_sh.at[six_v.at[step]], ssems[b],
                                 add=True)

        for b in range(2):
            pltpu.make_async_copy(tbl.at[gix_v.at[b]], gbufs[b],
                                  gsems[b]).wait()
            pltpu.make_async_copy(sbufs[b], acc_sh.at[six_v.at[b]],
                                  ssems[b]).wait()

    plsc.subcore_barrier()
    woff = (1 - c) * _NP
    pltpu.sync_copy(acc_sh.at[pl.ds(r0, _RPT)], out.at[pl.ds(woff + r0, _RPT)])


_spmm = functools.partial(
    pl.kernel,
    out_type=jax.ShapeDtypeStruct((2 * _NP, _D), _f32),
    mesh=plsc.VectorSubcoreMesh(core_axis_name="c", subcore_axis_name="s"),
    scratch_types=[
        pltpu.VMEM((_SUB, _CHUNK), _i32),
        pltpu.VMEM((_SUB, _CHUNK), _i32),
        pltpu.VMEM((_SUB, _CHUNK), _f32),
        pltpu.VMEM((_CHUNK, _D), _f32),
        pltpu.VMEM((_CHUNK, _D), _f32),
        pltpu.VMEM((_CHUNK, _D), _f32),
        pltpu.VMEM((_CHUNK, _D), _f32),
        pltpu.VMEM_SHARED((_NP, _D), _f32),
        pltpu.SemaphoreType.DMA,
        pltpu.SemaphoreType.DMA,
        pltpu.SemaphoreType.DMA,
        pltpu.SemaphoreType.DMA,
    ],
    compiler_params=pltpu.CompilerParams(needs_layout_passes=False),
)(_spmm_body)


_GROWS = 6 * _B
_GPW = _GROWS // (_NC * _NS)
_GCHUNK = 128
_GSTEPS = _GPW // _GCHUNK


def _gather_body(tbl, idx, out, idx_v, buf_v, sem):
    c = lax.axis_index("c")
    s = lax.axis_index("s")
    wid = s * _NC + c

    @pl.loop(0, _GSTEPS)
    def _step(k):
        base = wid * _GPW + k * _GCHUNK
        pltpu.sync_copy(idx.at[pl.ds(base, _GCHUNK)], idx_v)
        pltpu.async_copy(tbl.at[idx_v], buf_v, sem).wait()
        pltpu.sync_copy(buf_v, out.at[pl.ds(base, _GCHUNK)])


_gather = functools.partial(
    pl.kernel,
    out_type=jax.ShapeDtypeStruct((_GROWS, _D), _f32),
    mesh=plsc.VectorSubcoreMesh(core_axis_name="c", subcore_axis_name="s"),
    scratch_types=[
        pltpu.VMEM((_GCHUNK,), _i32),
        pltpu.VMEM((_GCHUNK, _D), _f32),
        pltpu.SemaphoreType.DMA,
    ],
)(_gather_body)


def _dense_small_body(eu0, ei0, zu1, zi1, ut, vt, wapi, bapi, wm, bm,
                      wu_out, wi_out, reg_out):
    wu_out[...] = jnp.dot(vt[...], ei0[...] + zi1[...],
                          preferred_element_type=_f32)
    wi_out[...] = jnp.dot(ut[...], eu0[...] + zu1[...],
                          preferred_element_type=_f32)
    reg = (jnp.sum(eu0[...] * eu0[...]) + jnp.sum(ei0[...] * ei0[...])
           + jnp.sum(wapi[...] * wapi[...]) + jnp.sum(bapi[...] * bapi[...])
           + jnp.sum(wm[...] * wm[...]) + jnp.sum(bm[...] * bm[...]))
    reg_out[...] = jnp.reshape(reg * _LAM2, (1, 1))


_dense_small = pl.pallas_call(
    _dense_small_body,
    out_shape=[
        jax.ShapeDtypeStruct((64, _D), _f32),
        jax.ShapeDtypeStruct((64, _D), _f32),
        jax.ShapeDtypeStruct((1, 1), _f32),
    ],
)


_ABLK = 1000


def _assemble_body(eu0, ei0, zu1, zi1, zu2, zi2, umul, vmul, wu, wi,
                   eu_out, ei_out, gu_out, gi_out):
    eu_out[...] = eu0[...] + zu1[...] + zu2[...]
    ei_out[...] = ei0[...] + zi1[...] + zi2[...]
    gu_out[...] = eu0[...] + jnp.dot(umul[...], wu[...],
                                     preferred_element_type=_f32)
    gi_out[...] = ei0[...] + jnp.dot(vmul[...], wi[...],
                                     preferred_element_type=_f32)


def _assemble(eu0, ei0, zu1, zi1, zu2, zi2, umul, vmul, wu, wi):
    blk = lambda w: pl.BlockSpec((_ABLK, w), lambda i: (i, 0))
    full = pl.BlockSpec((64, _D), lambda i: (0, 0))
    return pl.pallas_call(
        _assemble_body,
        grid=(_N // _ABLK,),
        in_specs=[blk(_D)] * 6 + [blk(64), blk(64), full, full],
        out_specs=[blk(_D)] * 4,
        out_shape=[jax.ShapeDtypeStruct((_N, _D), _f32)] * 4,
    )(eu0, ei0, zu1, zi1, zu2, zi2, umul, vmul, wu, wi)


_JB = 400
_NJ = _N // _JB


def _loss_body(eu_blk, ei_blk, gu_b, gi_b, eu_b, eip, ein, reg,
               loss_out, lr_out, ls_out, acc_u, acc_i):
    j = pl.program_id(0)

    @pl.when(j == 0)
    def _init():
        acc_u[...] = jnp.zeros_like(acc_u)
        acc_i[...] = jnp.zeros_like(acc_i)

    dn = (((1,), (1,)), ((), ()))
    mu = lax.dot_general(gu_b[...], eu_blk[...], dn,
                         preferred_element_type=_f32)
    acc_u[...] += jnp.sum(jnp.exp(mu / _TEMP), axis=1, keepdims=True)
    mi = lax.dot_general(gi_b[...], ei_blk[...], dn,
                         preferred_element_type=_f32)
    acc_i[...] += jnp.sum(jnp.exp(mi / _TEMP), axis=1, keepdims=True)

    @pl.when(j == _NJ - 1)
    def _final():
        neg_score = (jnp.mean(jnp.log(acc_u[...] + 1e-08))
                     + jnp.mean(jnp.log(acc_i[...] + 1e-08)))
        gu = gu_b[...]
        eu = eu_b[...]
        ei_cat = jnp.concatenate([eip[...], ein[...]], axis=0)
        pos_score = (jnp.mean(jnp.clip(jnp.sum(gu * eu, 1) / _TEMP, -5.0, 5.0))
                     + jnp.mean(jnp.clip(jnp.sum(gi_b[...] * ei_cat, 1) / _TEMP,
                                         -5.0, 5.0)))
        loss_s = -pos_score + neg_score
        ps = jnp.sum(eu * eip[...], 1)
        ns = jnp.sum(eu * ein[...], 1)
        loss_r = jnp.mean(jnp.log(1.0 + jnp.exp(ns - ps)))
        lam_ls = _LAM1 * loss_s
        ls_out[...] = jnp.reshape(lam_ls, (1, 1))
        lr_out[...] = jnp.reshape(loss_r, (1, 1))
        loss_out[...] = jnp.reshape(loss_r + lam_ls, (1, 1)) + reg[...]


def _loss(E_u, E_i, gu_b, gi_b, eu_b, eip, ein, reg):
    blk = pl.BlockSpec((_JB, _D), lambda j: (j, 0))
    fullb = lambda r: pl.BlockSpec((r, _D), lambda j: (0, 0))
    one = pl.BlockSpec((1, 1), lambda j: (0, 0))
    return pl.pallas_call(
        _loss_body,
        grid=(_NJ,),
        in_specs=[blk, blk, fullb(_B), fullb(2 * _B), fullb(_B), fullb(_B),
                  fullb(_B), one],
        out_specs=[one, one, one],
        out_shape=[jax.ShapeDtypeStruct((1, 1), _f32)] * 3,
        scratch_shapes=[pltpu.VMEM((_B, 1), _f32),
                        pltpu.VMEM((2 * _B, 1), _f32)],
    )(E_u, E_i, gu_b, gi_b, eu_b, eip, ein, reg)


def kernel(E_u_0, E_i_0, vals, ut, vt, u_mul_s, v_mul_s,
           pos_api_emb, neg_api_emb, mashup_emb,
           W_api, b_api, W_mashup, b_mashup,
           rows, cols, uids, pos, neg):
    rows = rows.astype(_i32)
    cols = cols.astype(_i32)

    def _slab(x, fill):
        x = x.reshape(_NS, _EPT)
        x = jnp.pad(x, ((0, 0), (0, _EPAD - _EPT)), constant_values=fill)
        return x.reshape(_NS, _NSTEP, _CHUNK)

    gidx = jnp.stack([_slab(cols, 0), _slab(rows, 0) + _NP])
    sidx = jnp.stack([_slab(rows, 0), _slab(cols, 0)])
    vals_t = _slab(vals, 0.0)
    zeros = jnp.zeros((_NP, _D), _f32)
    pad = jnp.zeros((_NP - _N, _D), _f32)

    T0 = jnp.concatenate([E_i_0, pad, E_u_0, pad], axis=0)
    Z1 = _spmm(T0, gidx, sidx, vals_t, zeros)
    Z2 = _spmm(Z1, gidx, sidx, vals_t, zeros)
    zi1, zu1 = Z1[:_N], Z1[_NP:_NP + _N]
    zi2, zu2 = Z2[:_N], Z2[_NP:_NP + _N]

    wu, wi, reg = _dense_small(E_u_0, E_i_0, zu1, zi1, ut, vt,
                               W_api, b_api.reshape(1, _D),
                               W_mashup, b_mashup.reshape(1, _D))
    E_u, E_i, G_u, G_i = _assemble(E_u_0, E_i_0, zu1, zi1, zu2, zi2,
                                   u_mul_s, v_mul_s, wu, wi)

    T4 = jnp.concatenate([G_u, E_u, G_i, E_i], axis=0)
    u32 = uids.astype(_i32)
    p32 = pos.astype(_i32)
    n32 = neg.astype(_i32)
    gidx2 = jnp.concatenate([u32, u32 + _N, p32 + 2 * _N, n32 + 2 * _N,
                             p32 + 3 * _N, n32 + 3 * _N])
    rows_g = _gather(T4, gidx2)
    gu_b = rows_g[:_B]
    eu_b = rows_g[_B:2 * _B]
    gi_b = rows_g[2 * _B:4 * _B]
    eip = rows_g[4 * _B:5 * _B]
    ein = rows_g[5 * _B:]

    loss, loss_r, lam_ls = _loss(E_u, E_i, gu_b, gi_b, eu_b, eip, ein, reg)
    return (loss.reshape(()), loss_r.reshape(()), lam_ls.reshape(()),
            mashup_emb, pos_api_emb, neg_api_emb, E_u, E_i)
